# baseline (speedup 1.0000x reference)
import functools

import jax
import jax.numpy as jnp
from jax import lax
from jax.experimental import pallas as pl
from jax.experimental.pallas import tpu as pltpu
from jax.experimental.pallas import tpu_sc as plsc

N = 10000
E = 320000
NC = 2
NS = 16
NW = NC * NS
C = 80
NCH = E // (NW * C)
STRIPE = 632
NPAD = NS * STRIPE
BNC = 1.0 / (1.0 + 1e-5) ** 0.5

_MESH = plsc.VectorSubcoreMesh(
    core_axis_name="c", subcore_axis_name="s", num_cores=NC, num_subcores=NS
)


def _zero_stripe(zbuf, acc, s, d, dt):
    lanes = 32 if dt == jnp.bfloat16 else 16
    z = jnp.zeros((lanes,), dt)

    def zrow(i, carry):
        for q in range(d // lanes):
            zbuf[i, pl.ds(q * lanes, lanes)] = z
        return carry

    lax.fori_loop(0, STRIPE, zrow, 0)
    pltpu.sync_copy(zbuf, acc.at[pl.ds(s * STRIPE, STRIPE)])


def _make_sc_scatter(d):
    @functools.partial(
        pl.kernel,
        out_type=jax.ShapeDtypeStruct((NC, NPAD, d), jnp.bfloat16),
        mesh=_MESH,
        compiler_params=pltpu.CompilerParams(use_tc_tiling_on_sc=False),
        scratch_types=[
            pltpu.VMEM((NCH, C), jnp.int32),
            pltpu.VMEM((NCH, C), jnp.int32),
            pltpu.VMEM((2, C, d), jnp.bfloat16),
            pltpu.VMEM((STRIPE, d), jnp.bfloat16),
            pltpu.VMEM_SHARED((NPAD, d), jnp.bfloat16),
            pltpu.VMEM_SHARED((NPAD, d), jnp.bfloat16),
            pltpu.SemaphoreType.DMA,
            pltpu.SemaphoreType.DMA,
        ],
    )
    def k(h_hbm, row_hbm, col_hbm, out_hbm, rowi, coli, rows, zbuf, acc,
          h_sp, sem_g, sem_s):
        c = lax.axis_index("c")
        s = lax.axis_index("s")
        wid = c * NS + s
        last = N // STRIPE
        rem_rows = N - last * STRIPE

        @pl.when(s < last)
        def _():
            pltpu.sync_copy(h_hbm.at[pl.ds(s * STRIPE, STRIPE)],
                            h_sp.at[pl.ds(s * STRIPE, STRIPE)])

        @pl.when(s == last)
        def _():
            pltpu.sync_copy(h_hbm.at[pl.ds(last * STRIPE, rem_rows)],
                            h_sp.at[pl.ds(last * STRIPE, rem_rows)])
        _zero_stripe(zbuf, acc, s, d, jnp.bfloat16)
        pltpu.sync_copy(row_hbm.at[wid], rowi)
        pltpu.sync_copy(col_hbm.at[wid], coli)
        plsc.subcore_barrier()

        pltpu.async_copy(h_sp.at[rowi.at[0]], rows.at[0], sem_g)

        def chunk(j, carry):
            b = lax.rem(j, 2)
            nb = lax.rem(j + 1, 2)
            pltpu.make_async_copy(h_sp.at[rowi.at[j]], rows.at[b], sem_g).wait()

            @pl.when(j >= 1)
            def _():
                pltpu.make_async_copy(
                    rows.at[nb], acc.at[coli.at[j - 1]], sem_s
                ).wait()

            @pl.when(j + 1 < NCH)
            def _():
                pltpu.async_copy(h_hbm.at[rowi.at[j + 1]], rows.at[nb], sem_g)

            pltpu.async_copy(rows.at[b], acc.at[coli.at[j]], sem_s, add=True)
            return carry

        lax.fori_loop(0, NCH, chunk, 0)
        pltpu.make_async_copy(
            rows.at[(NCH - 1) % 2], acc.at[coli.at[NCH - 1]], sem_s
        ).wait()
        plsc.subcore_barrier()
        pltpu.sync_copy(
            acc.at[pl.ds(s * STRIPE, STRIPE)],
            out_hbm.at[c, pl.ds(s * STRIPE, STRIPE)],
        )

    return k


def _make_sc_degree():
    d = 16

    @functools.partial(
        pl.kernel,
        out_type=jax.ShapeDtypeStruct((NC, NPAD, d), jnp.float32),
        mesh=_MESH,
        compiler_params=pltpu.CompilerParams(use_tc_tiling_on_sc=False),
        scratch_types=[
            pltpu.VMEM((NCH, C), jnp.int32),
            pltpu.VMEM((C, d), jnp.float32),
            pltpu.VMEM((STRIPE, d), jnp.float32),
            pltpu.VMEM_SHARED((NPAD, d), jnp.float32),
        ],
    )
    def k(col_hbm, out_hbm, coli, ones, zbuf, acc):
        c = lax.axis_index("c")
        s = lax.axis_index("s")
        wid = c * NS + s
        one = jnp.ones((16,), jnp.float32)

        def orow(i, carry):
            ones[i, pl.ds(0, 16)] = one
            return carry

        lax.fori_loop(0, C, orow, 0)
        _zero_stripe(zbuf, acc, s, d, jnp.float32)
        pltpu.sync_copy(col_hbm.at[wid], coli)
        plsc.subcore_barrier()

        def chunk(j, carry):
            pltpu.sync_copy(ones, acc.at[coli.at[j]], add=True)
            return carry

        lax.fori_loop(0, NCH, chunk, 0)
        plsc.subcore_barrier()
        pltpu.sync_copy(
            acc.at[pl.ds(s * STRIPE, STRIPE)],
            out_hbm.at[c, pl.ds(s * STRIPE, STRIPE)],
        )

    return k


_sc_degree = _make_sc_degree()
_sc_scatter32 = _make_sc_scatter(32)
_sc_scatter64 = _make_sc_scatter(64)


def _tc1(d0, d1, x, w1, b1):
    def body(d0_ref, d1_ref, x_ref, w_ref, b_ref, t_ref, dis_ref):
        deg = 1.0 + d0_ref[...] + d1_ref[...]
        dis = lax.rsqrt(deg)
        h = jnp.dot(x_ref[...], w_ref[...], preferred_element_type=jnp.float32)
        t_ref[...] = ((h + b_ref[...]) * dis).astype(jnp.bfloat16)
        dis_ref[...] = dis

    return pl.pallas_call(
        body,
        out_shape=(
            jax.ShapeDtypeStruct((N, 32), jnp.bfloat16),
            jax.ShapeDtypeStruct((N, 1), jnp.float32),
        ),
    )(d0, d1, x, w1, b1)


def _tc_mid(s0, s1, t, dis, g, be, w, b, d_out, scale_out):
    def body(s0_ref, s1_ref, t_ref, dis_ref, g_ref, be_ref, w_ref, b_ref, o_ref):
        f32 = jnp.float32
        h = (s0_ref[...].astype(f32) + s1_ref[...].astype(f32)
             + t_ref[...].astype(f32)) * dis_ref[...]
        u = jnp.maximum(h * (g_ref[...] * BNC) + be_ref[...], 0.0)
        o = jnp.dot(u, w_ref[...], preferred_element_type=jnp.float32) + b_ref[...]
        if scale_out:
            o = (o * dis_ref[...]).astype(jnp.bfloat16)
        else:
            o = jnp.maximum(o, 0.0)
        o_ref[...] = o

    return pl.pallas_call(
        body,
        out_shape=jax.ShapeDtypeStruct(
            (N, d_out), jnp.bfloat16 if scale_out else jnp.float32),
    )(s0, s1, t, dis, g, be, w, b)


def kernel(x, edge_index, W1, b1, g1, be1, W2, b2, g2, be2, W3, b3):
    row_p = edge_index[0].reshape(NW, NCH, C)
    col_p = edge_index[1].reshape(NW, NCH, C)

    degp = _sc_degree(col_p)
    d0 = degp[0, :N, 0:1]
    d1 = degp[1, :N, 0:1]

    t1, dis = _tc1(d0, d1, x, W1, b1.reshape(1, 32))
    s1 = _sc_scatter32(t1, row_p, col_p)
    t2 = _tc_mid(s1[0, :N], s1[1, :N], t1, dis, g1.reshape(1, 32),
                 be1.reshape(1, 32), W2, b2.reshape(1, 64), 64, True)
    s2 = _sc_scatter64(t2, row_p, col_p)
    out = _tc_mid(s2[0, :N], s2[1, :N], t2, dis, g2.reshape(1, 64),
                  be2.reshape(1, 64), W3, b3.reshape(1, 128), 128, False)
    return out


# speedup vs baseline: 1.3835x; 1.3835x over previous
import functools

import jax
import jax.numpy as jnp
from jax import lax
from jax.experimental import pallas as pl
from jax.experimental.pallas import tpu as pltpu
from jax.experimental.pallas import tpu_sc as plsc

N = 10000
E = 320000
NC = 2
NS = 16
NW = NC * NS
C = 80
NCH = E // (NW * C)
STRIPE = 632
NPAD = NS * STRIPE
BNC = 1.0 / (1.0 + 1e-5) ** 0.5

_MESH = plsc.VectorSubcoreMesh(
    core_axis_name="c", subcore_axis_name="s", num_cores=NC, num_subcores=NS
)


def _zero_stripe(zbuf, acc, s, d, dt):
    lanes = 32 if dt == jnp.bfloat16 else 16
    z = jnp.zeros((lanes,), dt)

    def zrow(i, carry):
        for q in range(d // lanes):
            zbuf[i, pl.ds(q * lanes, lanes)] = z
        return carry

    lax.fori_loop(0, STRIPE, zrow, 0)
    pltpu.sync_copy(zbuf, acc.at[pl.ds(s * STRIPE, STRIPE)])


def _make_sc_scatter(d):
    @functools.partial(
        pl.kernel,
        out_type=jax.ShapeDtypeStruct((NC, NPAD, d), jnp.bfloat16),
        mesh=_MESH,
        compiler_params=pltpu.CompilerParams(use_tc_tiling_on_sc=False),
        scratch_types=[
            pltpu.VMEM((NCH, C), jnp.int32),
            pltpu.VMEM((NCH, C), jnp.int32),
            pltpu.VMEM((4, C, d), jnp.bfloat16),
            pltpu.VMEM((STRIPE, d), jnp.bfloat16),
            pltpu.VMEM_SHARED((NPAD, d), jnp.bfloat16),
            pltpu.SemaphoreType.DMA,
            pltpu.SemaphoreType.DMA,
            pltpu.SemaphoreType.DMA,
            pltpu.SemaphoreType.DMA,
        ],
    )
    def k(h_hbm, row_hbm, col_hbm, out_hbm, rowi, coli, rows, zbuf, acc,
          sem_ge, sem_go, sem_se, sem_so):
        c = lax.axis_index("c")
        s = lax.axis_index("s")
        wid = c * NS + s
        _zero_stripe(zbuf, acc, s, d, jnp.bfloat16)
        pltpu.sync_copy(row_hbm.at[wid], rowi)
        pltpu.sync_copy(col_hbm.at[wid], coli)
        plsc.subcore_barrier()

        pltpu.async_copy(h_hbm.at[rowi.at[0]], rows.at[0], sem_ge)
        pltpu.async_copy(h_hbm.at[rowi.at[1]], rows.at[1], sem_go)

        def halfstep(j, sem_g, sem_s, i):
            bj = lax.rem(j, 4)
            bn = lax.rem(j + 2, 4)
            pltpu.make_async_copy(h_hbm.at[rowi.at[j]], rows.at[bj], sem_g).wait()

            @pl.when(i >= 1)
            def _():
                pltpu.make_async_copy(
                    rows.at[bn], acc.at[coli.at[j - 2]], sem_s
                ).wait()

            @pl.when(j + 2 < NCH)
            def _():
                pltpu.async_copy(h_hbm.at[rowi.at[j + 2]], rows.at[bn], sem_g)

            pltpu.async_copy(rows.at[bj], acc.at[coli.at[j]], sem_s, add=True)

        def pair(i, carry):
            halfstep(2 * i, sem_ge, sem_se, i)
            halfstep(2 * i + 1, sem_go, sem_so, i)
            return carry

        lax.fori_loop(0, NCH // 2, pair, 0)
        t = NCH - 1
        bt = lax.rem(t, 4)
        pltpu.make_async_copy(h_hbm.at[rowi.at[t]], rows.at[bt], sem_ge).wait()
        pltpu.async_copy(rows.at[bt], acc.at[coli.at[t]], sem_se, add=True)
        pltpu.make_async_copy(
            rows.at[lax.rem(t + 2, 4)], acc.at[coli.at[t - 2]], sem_se).wait()
        pltpu.make_async_copy(rows.at[bt], acc.at[coli.at[t]], sem_se).wait()
        pltpu.make_async_copy(
            rows.at[lax.rem(t + 1, 4)], acc.at[coli.at[t - 1]], sem_so).wait()
        plsc.subcore_barrier()
        pltpu.sync_copy(
            acc.at[pl.ds(s * STRIPE, STRIPE)],
            out_hbm.at[c, pl.ds(s * STRIPE, STRIPE)],
        )

    return k


def _make_sc_degree():
    d = 16

    @functools.partial(
        pl.kernel,
        out_type=jax.ShapeDtypeStruct((NC, NPAD, d), jnp.float32),
        mesh=_MESH,
        compiler_params=pltpu.CompilerParams(use_tc_tiling_on_sc=False),
        scratch_types=[
            pltpu.VMEM((NCH, C), jnp.int32),
            pltpu.VMEM((C, d), jnp.float32),
            pltpu.VMEM((STRIPE, d), jnp.float32),
            pltpu.VMEM_SHARED((NPAD, d), jnp.float32),
            pltpu.SemaphoreType.DMA,
        ],
    )
    def k(col_hbm, out_hbm, coli, ones, zbuf, acc, sem_s):
        c = lax.axis_index("c")
        s = lax.axis_index("s")
        wid = c * NS + s
        one = jnp.ones((16,), jnp.float32)

        def orow(i, carry):
            ones[i, pl.ds(0, 16)] = one
            return carry

        lax.fori_loop(0, C, orow, 0)
        _zero_stripe(zbuf, acc, s, d, jnp.float32)
        pltpu.sync_copy(col_hbm.at[wid], coli)
        plsc.subcore_barrier()

        def group(g, carry):
            for u in range(5):
                pltpu.async_copy(
                    ones, acc.at[coli.at[5 * g + u]], sem_s, add=True)
            for u in range(5):
                pltpu.make_async_copy(
                    ones, acc.at[coli.at[5 * g + u]], sem_s).wait()
            return carry

        lax.fori_loop(0, NCH // 5, group, 0)
        plsc.subcore_barrier()
        pltpu.sync_copy(
            acc.at[pl.ds(s * STRIPE, STRIPE)],
            out_hbm.at[c, pl.ds(s * STRIPE, STRIPE)],
        )

    return k


_sc_degree = _make_sc_degree()
_sc_scatter32 = _make_sc_scatter(32)
_sc_scatter64 = _make_sc_scatter(64)


def _tc1(d0, d1, x, w1, b1):
    def body(d0_ref, d1_ref, x_ref, w_ref, b_ref, t_ref, dis_ref):
        deg = 1.0 + d0_ref[...] + d1_ref[...]
        dis = lax.rsqrt(deg)
        h = jnp.dot(x_ref[...], w_ref[...], preferred_element_type=jnp.float32)
        t_ref[...] = ((h + b_ref[...]) * dis).astype(jnp.bfloat16)
        dis_ref[...] = dis

    return pl.pallas_call(
        body,
        out_shape=(
            jax.ShapeDtypeStruct((N, 32), jnp.bfloat16),
            jax.ShapeDtypeStruct((N, 1), jnp.float32),
        ),
    )(d0, d1, x, w1, b1)


def _tc_mid(s0, s1, t, dis, g, be, w, b, d_out, scale_out):
    def body(s0_ref, s1_ref, t_ref, dis_ref, g_ref, be_ref, w_ref, b_ref, o_ref):
        f32 = jnp.float32
        h = (s0_ref[...].astype(f32) + s1_ref[...].astype(f32)
             + t_ref[...].astype(f32)) * dis_ref[...]
        u = jnp.maximum(h * (g_ref[...] * BNC) + be_ref[...], 0.0)
        o = jnp.dot(u, w_ref[...], preferred_element_type=jnp.float32) + b_ref[...]
        if scale_out:
            o = (o * dis_ref[...]).astype(jnp.bfloat16)
        else:
            o = jnp.maximum(o, 0.0)
        o_ref[...] = o

    return pl.pallas_call(
        body,
        out_shape=jax.ShapeDtypeStruct(
            (N, d_out), jnp.bfloat16 if scale_out else jnp.float32),
    )(s0, s1, t, dis, g, be, w, b)


def kernel(x, edge_index, W1, b1, g1, be1, W2, b2, g2, be2, W3, b3):
    row_p = edge_index[0].reshape(NW, NCH, C)
    col_p = edge_index[1].reshape(NW, NCH, C)

    degp = _sc_degree(col_p)
    d0 = degp[0, :N, 0:1]
    d1 = degp[1, :N, 0:1]

    t1, dis = _tc1(d0, d1, x, W1, b1.reshape(1, 32))
    s1 = _sc_scatter32(t1, row_p, col_p)
    t2 = _tc_mid(s1[0, :N], s1[1, :N], t1, dis, g1.reshape(1, 32),
                 be1.reshape(1, 32), W2, b2.reshape(1, 64), 64, True)
    s2 = _sc_scatter64(t2, row_p, col_p)
    out = _tc_mid(s2[0, :N], s2[1, :N], t2, dis, g2.reshape(1, 64),
                  be2.reshape(1, 64), W3, b3.reshape(1, 128), 128, False)
    return out


# speedup vs baseline: 1.4942x; 1.0800x over previous
import functools

import jax
import jax.numpy as jnp
from jax import lax
from jax.experimental import pallas as pl
from jax.experimental.pallas import tpu as pltpu
from jax.experimental.pallas import tpu_sc as plsc

N = 10000
E = 320000
NC = 2
NS = 16
NW = NC * NS
C = 80
NCH = E // (NW * C)
STRIPE = 632
NPAD = NS * STRIPE
BNC = 1.0 / (1.0 + 1e-5) ** 0.5

_MESH = plsc.VectorSubcoreMesh(
    core_axis_name="c", subcore_axis_name="s", num_cores=NC, num_subcores=NS
)


def _zero_stripe(zbuf, acc, s, d, dt):
    lanes = 32 if dt == jnp.bfloat16 else 16
    z = jnp.zeros((lanes,), dt)

    def zrow(i, carry):
        for q in range(d // lanes):
            zbuf[i, pl.ds(q * lanes, lanes)] = z
        return carry

    lax.fori_loop(0, STRIPE, zrow, 0)
    pltpu.sync_copy(zbuf, acc.at[pl.ds(s * STRIPE, STRIPE)])


def _make_sc_scatter(d):
    @functools.partial(
        pl.kernel,
        out_type=jax.ShapeDtypeStruct((NC, NPAD, d), jnp.bfloat16),
        mesh=_MESH,
        compiler_params=pltpu.CompilerParams(use_tc_tiling_on_sc=False),
        scratch_types=[
            pltpu.VMEM((NCH, C), jnp.int32),
            pltpu.VMEM((NCH, C), jnp.int32),
            pltpu.VMEM((4, C, d), jnp.bfloat16),
            pltpu.VMEM((STRIPE, d), jnp.bfloat16),
            pltpu.VMEM_SHARED((NPAD, d), jnp.bfloat16),
            pltpu.SemaphoreType.DMA,
            pltpu.SemaphoreType.DMA,
            pltpu.SemaphoreType.DMA,
            pltpu.SemaphoreType.DMA,
        ],
    )
    def k(h_hbm, row_hbm, col_hbm, out_hbm, rowi, coli, rows, zbuf, acc,
          sem_ge, sem_go, sem_se, sem_so):
        c = lax.axis_index("c")
        s = lax.axis_index("s")
        wid = c * NS + s
        _zero_stripe(zbuf, acc, s, d, jnp.bfloat16)
        pltpu.sync_copy(row_hbm.at[wid], rowi)
        pltpu.sync_copy(col_hbm.at[wid], coli)
        plsc.subcore_barrier()

        pltpu.async_copy(h_hbm.at[rowi.at[0]], rows.at[0], sem_ge)
        pltpu.async_copy(h_hbm.at[rowi.at[1]], rows.at[1], sem_go)

        def halfstep(j, sem_g, sem_s, i):
            bj = lax.rem(j, 4)
            bn = lax.rem(j + 2, 4)
            pltpu.make_async_copy(h_hbm.at[rowi.at[j]], rows.at[bj], sem_g).wait()

            @pl.when(i >= 1)
            def _():
                pltpu.make_async_copy(
                    rows.at[bn], acc.at[coli.at[j - 2]], sem_s
                ).wait()

            @pl.when(j + 2 < NCH)
            def _():
                pltpu.async_copy(h_hbm.at[rowi.at[j + 2]], rows.at[bn], sem_g)

            pltpu.async_copy(rows.at[bj], acc.at[coli.at[j]], sem_s, add=True)

        def pair(i, carry):
            halfstep(2 * i, sem_ge, sem_se, i)
            halfstep(2 * i + 1, sem_go, sem_so, i)
            return carry

        lax.fori_loop(0, NCH // 2, pair, 0)
        t = NCH - 1
        bt = lax.rem(t, 4)
        pltpu.make_async_copy(h_hbm.at[rowi.at[t]], rows.at[bt], sem_ge).wait()
        pltpu.async_copy(rows.at[bt], acc.at[coli.at[t]], sem_se, add=True)
        pltpu.make_async_copy(
            rows.at[lax.rem(t + 2, 4)], acc.at[coli.at[t - 2]], sem_se).wait()
        pltpu.make_async_copy(rows.at[bt], acc.at[coli.at[t]], sem_se).wait()
        pltpu.make_async_copy(
            rows.at[lax.rem(t + 1, 4)], acc.at[coli.at[t - 1]], sem_so).wait()
        plsc.subcore_barrier()
        pltpu.sync_copy(
            acc.at[pl.ds(s * STRIPE, STRIPE)],
            out_hbm.at[c, pl.ds(s * STRIPE, STRIPE)],
        )

    return k


def _make_sc_degree():
    d = 16

    @functools.partial(
        pl.kernel,
        out_type=jax.ShapeDtypeStruct((NC, NPAD, d), jnp.float32),
        mesh=_MESH,
        compiler_params=pltpu.CompilerParams(use_tc_tiling_on_sc=False),
        scratch_types=[
            pltpu.VMEM((NCH, C), jnp.int32),
            pltpu.VMEM((C, d), jnp.float32),
            pltpu.VMEM((STRIPE, d), jnp.float32),
            pltpu.VMEM_SHARED((NPAD, d), jnp.float32),
            pltpu.SemaphoreType.DMA,
        ],
    )
    def k(col_hbm, out_hbm, coli, ones, zbuf, acc, sem_s):
        c = lax.axis_index("c")
        s = lax.axis_index("s")
        wid = c * NS + s
        one = jnp.ones((16,), jnp.float32)

        def orow(i, carry):
            ones[i, pl.ds(0, 16)] = one
            return carry

        lax.fori_loop(0, C, orow, 0)
        _zero_stripe(zbuf, acc, s, d, jnp.float32)
        pltpu.sync_copy(col_hbm.at[wid], coli)
        plsc.subcore_barrier()

        def group(g, carry):
            for u in range(5):
                pltpu.async_copy(
                    ones, acc.at[coli.at[5 * g + u]], sem_s, add=True)
            for u in range(5):
                pltpu.make_async_copy(
                    ones, acc.at[coli.at[5 * g + u]], sem_s).wait()
            return carry

        lax.fori_loop(0, NCH // 5, group, 0)
        plsc.subcore_barrier()
        pltpu.sync_copy(
            acc.at[pl.ds(s * STRIPE, STRIPE)],
            out_hbm.at[c, pl.ds(s * STRIPE, STRIPE)],
        )

    return k


_sc_degree = _make_sc_degree()
_sc_scatter32 = _make_sc_scatter(32)
_sc_scatter64 = _make_sc_scatter(64)


def _tc1(degp, x, w1, b1):
    def body(dp_ref, x_ref, w_ref, b_ref, t_ref, dis_ref):
        deg = 1.0 + dp_ref[0, :N, 0:1] + dp_ref[1, :N, 0:1]
        dis = lax.rsqrt(deg)
        h = jnp.dot(x_ref[...], w_ref[...], preferred_element_type=jnp.float32)
        t_ref[...] = ((h + b_ref[...]) * dis).astype(jnp.bfloat16)
        dis_ref[...] = dis

    return pl.pallas_call(
        body,
        out_shape=(
            jax.ShapeDtypeStruct((N, 32), jnp.bfloat16),
            jax.ShapeDtypeStruct((N, 1), jnp.float32),
        ),
    )(degp, x, w1, b1)


def _tc_mid(sp, t, dis, g, be, w, b, d_out, scale_out):
    def body(sp_ref, t_ref, dis_ref, g_ref, be_ref, w_ref, b_ref, o_ref):
        f32 = jnp.float32
        h = (sp_ref[0, :N].astype(f32) + sp_ref[1, :N].astype(f32)
             + t_ref[...].astype(f32)) * dis_ref[...]
        u = jnp.maximum(h * (g_ref[...] * BNC) + be_ref[...], 0.0)
        o = jnp.dot(u, w_ref[...], preferred_element_type=jnp.float32) + b_ref[...]
        if scale_out:
            o = (o * dis_ref[...]).astype(jnp.bfloat16)
        else:
            o = jnp.maximum(o, 0.0)
        o_ref[...] = o

    return pl.pallas_call(
        body,
        out_shape=jax.ShapeDtypeStruct(
            (N, d_out), jnp.bfloat16 if scale_out else jnp.float32),
    )(sp, t, dis, g, be, w, b)


def kernel(x, edge_index, W1, b1, g1, be1, W2, b2, g2, be2, W3, b3):
    row_p = edge_index[0].reshape(NW, NCH, C)
    col_p = edge_index[1].reshape(NW, NCH, C)

    degp = _sc_degree(col_p)
    t1, dis = _tc1(degp, x, W1, b1.reshape(1, 32))
    s1 = _sc_scatter32(t1, row_p, col_p)
    t2 = _tc_mid(s1, t1, dis, g1.reshape(1, 32),
                 be1.reshape(1, 32), W2, b2.reshape(1, 64), 64, True)
    s2 = _sc_scatter64(t2, row_p, col_p)
    out = _tc_mid(s2, t2, dis, g2.reshape(1, 64),
                  be2.reshape(1, 64), W3, b3.reshape(1, 128), 128, False)
    return out


# speedup vs baseline: 1.8701x; 1.2516x over previous
import functools

import jax
import jax.numpy as jnp
from jax import lax
from jax.experimental import pallas as pl
from jax.experimental.pallas import tpu as pltpu
from jax.experimental.pallas import tpu_sc as plsc

N = 10000
E = 320000
NC = 2
NS = 16
NW = NC * NS
C = 80
NCH = E // (NW * C)
STRIPE = 632
NPAD = NS * STRIPE
BNC = 1.0 / (1.0 + 1e-5) ** 0.5

_MESH = plsc.VectorSubcoreMesh(
    core_axis_name="c", subcore_axis_name="s", num_cores=NC, num_subcores=NS
)


def _zero_stripe(zbuf, acc, s, d, dt):
    lanes = 32 if dt == jnp.bfloat16 else 16
    z = jnp.zeros((lanes,), dt)

    def zrow(i, carry):
        for q in range(d // lanes):
            zbuf[i, pl.ds(q * lanes, lanes)] = z
        return carry

    lax.fori_loop(0, STRIPE, zrow, 0)
    pltpu.sync_copy(zbuf, acc.at[pl.ds(s * STRIPE, STRIPE)])


def _make_sc_scatter(d):
    @functools.partial(
        pl.kernel,
        out_type=jax.ShapeDtypeStruct((NC, NPAD, d), jnp.bfloat16),
        mesh=_MESH,
        compiler_params=pltpu.CompilerParams(use_tc_tiling_on_sc=False),
        scratch_types=[
            pltpu.VMEM((NCH, C), jnp.int32),
            pltpu.VMEM((NCH, C), jnp.int32),
            pltpu.VMEM((10, C, d), jnp.bfloat16),
            pltpu.VMEM((STRIPE, d), jnp.bfloat16),
            pltpu.VMEM_SHARED((NPAD, d), jnp.bfloat16),
        ] + [pltpu.SemaphoreType.DMA] * 10,
    )
    def k(h_hbm, row_hbm, col_hbm, out_hbm, rowi, coli, rows, zbuf, acc,
          *sems):
        gs = sems[:5]
        ss = sems[5:]
        c = lax.axis_index("c")
        s = lax.axis_index("s")
        wid = c * NS + s
        _zero_stripe(zbuf, acc, s, d, jnp.bfloat16)
        pltpu.sync_copy(row_hbm.at[wid], rowi)
        pltpu.sync_copy(col_hbm.at[wid], coli)
        plsc.subcore_barrier()

        for u in range(5):
            pltpu.async_copy(h_hbm.at[rowi.at[u]], rows.at[u], gs[u])

        def block(i, carry):
            j0 = 5 * i
            for u in range(5):
                j = j0 + u
                bj = lax.rem(j, 10)
                bn = lax.rem(j + 5, 10)
                pltpu.make_async_copy(
                    h_hbm.at[rowi.at[j]], rows.at[bj], gs[u]).wait()

                @pl.when(i >= 1)
                def _():
                    pltpu.make_async_copy(
                        rows.at[bn], acc.at[coli.at[j - 5]], ss[u]).wait()

                @pl.when(j + 5 < NCH)
                def _():
                    pltpu.async_copy(
                        h_hbm.at[rowi.at[j + 5]], rows.at[bn], gs[u])

                pltpu.async_copy(rows.at[bj], acc.at[coli.at[j]], ss[u],
                                 add=True)
            return carry

        lax.fori_loop(0, NCH // 5, block, 0)
        for u in range(5):
            j = NCH - 5 + u
            pltpu.make_async_copy(
                rows.at[lax.rem(j, 10)], acc.at[coli.at[j]], ss[u]).wait()
        plsc.subcore_barrier()
        pltpu.sync_copy(
            acc.at[pl.ds(s * STRIPE, STRIPE)],
            out_hbm.at[c, pl.ds(s * STRIPE, STRIPE)],
        )

    return k


def _make_sc_degree():
    d = 16

    @functools.partial(
        pl.kernel,
        out_type=jax.ShapeDtypeStruct((NC, NPAD, d), jnp.float32),
        mesh=_MESH,
        compiler_params=pltpu.CompilerParams(use_tc_tiling_on_sc=False),
        scratch_types=[
            pltpu.VMEM((NCH, C), jnp.int32),
            pltpu.VMEM((C, d), jnp.float32),
            pltpu.VMEM((STRIPE, d), jnp.float32),
            pltpu.VMEM_SHARED((NPAD, d), jnp.float32),
            pltpu.SemaphoreType.DMA,
        ],
    )
    def k(col_hbm, out_hbm, coli, ones, zbuf, acc, sem_s):
        c = lax.axis_index("c")
        s = lax.axis_index("s")
        wid = c * NS + s
        one = jnp.ones((16,), jnp.float32)

        def orow(i, carry):
            ones[i, pl.ds(0, 16)] = one
            return carry

        lax.fori_loop(0, C, orow, 0)
        _zero_stripe(zbuf, acc, s, d, jnp.float32)
        pltpu.sync_copy(col_hbm.at[wid], coli)
        plsc.subcore_barrier()

        def group(g, carry):
            for u in range(5):
                pltpu.async_copy(
                    ones, acc.at[coli.at[5 * g + u]], sem_s, add=True)
            for u in range(5):
                pltpu.make_async_copy(
                    ones, acc.at[coli.at[5 * g + u]], sem_s).wait()
            return carry

        lax.fori_loop(0, NCH // 5, group, 0)
        plsc.subcore_barrier()
        pltpu.sync_copy(
            acc.at[pl.ds(s * STRIPE, STRIPE)],
            out_hbm.at[c, pl.ds(s * STRIPE, STRIPE)],
        )

    return k


_sc_degree = _make_sc_degree()
_sc_scatter32 = _make_sc_scatter(32)
_sc_scatter64 = _make_sc_scatter(64)


def _tc1(degp, x, w1, b1):
    def body(dp_ref, x_ref, w_ref, b_ref, t_ref, dis_ref):
        deg = 1.0 + dp_ref[0, :N, 0:1] + dp_ref[1, :N, 0:1]
        dis = lax.rsqrt(deg)
        h = jnp.dot(x_ref[...], w_ref[...], preferred_element_type=jnp.float32)
        t_ref[...] = ((h + b_ref[...]) * dis).astype(jnp.bfloat16)
        dis_ref[...] = dis

    return pl.pallas_call(
        body,
        out_shape=(
            jax.ShapeDtypeStruct((N, 32), jnp.bfloat16),
            jax.ShapeDtypeStruct((N, 1), jnp.float32),
        ),
    )(degp, x, w1, b1)


def _tc_mid(sp, t, dis, g, be, w, b, d_out, scale_out):
    def body(sp_ref, t_ref, dis_ref, g_ref, be_ref, w_ref, b_ref, o_ref):
        f32 = jnp.float32
        h = (sp_ref[0, :N].astype(f32) + sp_ref[1, :N].astype(f32)
             + t_ref[...].astype(f32)) * dis_ref[...]
        u = jnp.maximum(h * (g_ref[...] * BNC) + be_ref[...], 0.0)
        o = jnp.dot(u, w_ref[...], preferred_element_type=jnp.float32) + b_ref[...]
        if scale_out:
            o = (o * dis_ref[...]).astype(jnp.bfloat16)
        else:
            o = jnp.maximum(o, 0.0)
        o_ref[...] = o

    return pl.pallas_call(
        body,
        out_shape=jax.ShapeDtypeStruct(
            (N, d_out), jnp.bfloat16 if scale_out else jnp.float32),
    )(sp, t, dis, g, be, w, b)


def kernel(x, edge_index, W1, b1, g1, be1, W2, b2, g2, be2, W3, b3):
    row_p = edge_index[0].reshape(NW, NCH, C)
    col_p = edge_index[1].reshape(NW, NCH, C)

    degp = _sc_degree(col_p)
    t1, dis = _tc1(degp, x, W1, b1.reshape(1, 32))
    s1 = _sc_scatter32(t1, row_p, col_p)
    t2 = _tc_mid(s1, t1, dis, g1.reshape(1, 32),
                 be1.reshape(1, 32), W2, b2.reshape(1, 64), 64, True)
    s2 = _sc_scatter64(t2, row_p, col_p)
    out = _tc_mid(s2, t2, dis, g2.reshape(1, 64),
                  be2.reshape(1, 64), W3, b3.reshape(1, 128), 128, False)
    return out


# speedup vs baseline: 1.8750x; 1.0026x over previous
import functools

import jax
import jax.numpy as jnp
from jax import lax
from jax.experimental import pallas as pl
from jax.experimental.pallas import tpu as pltpu
from jax.experimental.pallas import tpu_sc as plsc

N = 10000
E = 320000
NC = 2
NS = 16
NW = NC * NS
C = 80
NCH = E // (NW * C)
STRIPE = 632
NPAD = NS * STRIPE
BNC = 1.0 / (1.0 + 1e-5) ** 0.5

_MESH = plsc.VectorSubcoreMesh(
    core_axis_name="c", subcore_axis_name="s", num_cores=NC, num_subcores=NS
)


def _zero_stripe(zbuf, acc, s, d, dt):
    lanes = 32 if dt == jnp.bfloat16 else 16
    z = jnp.zeros((lanes,), dt)

    def zrow(i, carry):
        for q in range(d // lanes):
            zbuf[i, pl.ds(q * lanes, lanes)] = z
        return carry

    lax.fori_loop(0, STRIPE, zrow, 0)
    pltpu.sync_copy(zbuf, acc.at[pl.ds(s * STRIPE, STRIPE)])


def _make_sc_scatter(d):
    @functools.partial(
        pl.kernel,
        out_type=jax.ShapeDtypeStruct((NC, NPAD, d), jnp.bfloat16),
        mesh=_MESH,
        compiler_params=pltpu.CompilerParams(use_tc_tiling_on_sc=False),
        scratch_types=[
            pltpu.VMEM((NCH, C), jnp.int32),
            pltpu.VMEM((NCH, C), jnp.int32),
            pltpu.VMEM((10, C, d), jnp.bfloat16),
            pltpu.VMEM((STRIPE, d), jnp.bfloat16),
            pltpu.VMEM_SHARED((NPAD, d), jnp.bfloat16),
        ] + [pltpu.SemaphoreType.DMA] * 10,
    )
    def k(h_hbm, row_hbm, col_hbm, out_hbm, rowi, coli, rows, zbuf, acc,
          *sems):
        gs = sems[:5]
        ss = sems[5:]
        c = lax.axis_index("c")
        s = lax.axis_index("s")
        wid = c * NS + s
        _zero_stripe(zbuf, acc, s, d, jnp.bfloat16)
        pltpu.sync_copy(row_hbm.at[wid], rowi)
        pltpu.sync_copy(col_hbm.at[wid], coli)
        plsc.subcore_barrier()

        for u in range(5):
            pltpu.async_copy(h_hbm.at[rowi.at[u]], rows.at[u], gs[u])

        def block(i, carry):
            j0 = 5 * i
            for u in range(5):
                j = j0 + u
                bj = lax.rem(j, 10)
                bn = lax.rem(j + 5, 10)
                pltpu.make_async_copy(
                    h_hbm.at[rowi.at[j]], rows.at[bj], gs[u]).wait()

                @pl.when(i >= 1)
                def _():
                    pltpu.make_async_copy(
                        rows.at[bn], acc.at[coli.at[j - 5]], ss[u]).wait()

                @pl.when(j + 5 < NCH)
                def _():
                    pltpu.async_copy(
                        h_hbm.at[rowi.at[j + 5]], rows.at[bn], gs[u])

                pltpu.async_copy(rows.at[bj], acc.at[coli.at[j]], ss[u],
                                 add=True)
            return carry

        lax.fori_loop(0, NCH // 5, block, 0)
        for u in range(5):
            j = NCH - 5 + u
            pltpu.make_async_copy(
                rows.at[lax.rem(j, 10)], acc.at[coli.at[j]], ss[u]).wait()
        plsc.subcore_barrier()
        pltpu.sync_copy(
            acc.at[pl.ds(s * STRIPE, STRIPE)],
            out_hbm.at[c, pl.ds(s * STRIPE, STRIPE)],
        )

    return k


def _make_sc_degree():
    d = 16

    @functools.partial(
        pl.kernel,
        out_type=jax.ShapeDtypeStruct((NC, NPAD, d), jnp.float32),
        mesh=_MESH,
        compiler_params=pltpu.CompilerParams(use_tc_tiling_on_sc=False),
        scratch_types=[
            pltpu.VMEM((NCH, C), jnp.int32),
            pltpu.VMEM((C, d), jnp.float32),
            pltpu.VMEM((STRIPE, d), jnp.float32),
            pltpu.VMEM_SHARED((NPAD, d), jnp.float32),
        ] + [pltpu.SemaphoreType.DMA] * 5,
    )
    def k(col_hbm, out_hbm, coli, ones, zbuf, acc, *sem_s):
        c = lax.axis_index("c")
        s = lax.axis_index("s")
        wid = c * NS + s
        one = jnp.ones((16,), jnp.float32)

        def orow(i, carry):
            ones[i, pl.ds(0, 16)] = one
            return carry

        lax.fori_loop(0, C, orow, 0)
        _zero_stripe(zbuf, acc, s, d, jnp.float32)
        pltpu.sync_copy(col_hbm.at[wid], coli)
        plsc.subcore_barrier()

        for u in range(5):
            pltpu.async_copy(ones, acc.at[coli.at[u]], sem_s[u], add=True)

        def group(g, carry):
            j0 = 5 * g
            for u in range(5):
                j = j0 + u
                pltpu.make_async_copy(ones, acc.at[coli.at[j]], sem_s[u]).wait()

                @pl.when(j + 5 < NCH)
                def _():
                    pltpu.async_copy(
                        ones, acc.at[coli.at[j + 5]], sem_s[u], add=True)
            return carry

        lax.fori_loop(0, NCH // 5, group, 0)
        plsc.subcore_barrier()
        pltpu.sync_copy(
            acc.at[pl.ds(s * STRIPE, STRIPE)],
            out_hbm.at[c, pl.ds(s * STRIPE, STRIPE)],
        )

    return k


_sc_degree = _make_sc_degree()
_sc_scatter32 = _make_sc_scatter(32)
_sc_scatter64 = _make_sc_scatter(64)


def _tc1(degp, x, w1, b1):
    def body(dp_ref, x_ref, w_ref, b_ref, t_ref, dis_ref):
        deg = 1.0 + dp_ref[0, :N, 0:1] + dp_ref[1, :N, 0:1]
        dis = lax.rsqrt(deg)
        h = jnp.dot(x_ref[...], w_ref[...], preferred_element_type=jnp.float32)
        t_ref[...] = ((h + b_ref[...]) * dis).astype(jnp.bfloat16)
        dis_ref[...] = dis

    return pl.pallas_call(
        body,
        out_shape=(
            jax.ShapeDtypeStruct((N, 32), jnp.bfloat16),
            jax.ShapeDtypeStruct((N, 1), jnp.float32),
        ),
    )(degp, x, w1, b1)


def _tc_mid(sp, t, dis, g, be, w, b, d_out, scale_out):
    def body(sp_ref, t_ref, dis_ref, g_ref, be_ref, w_ref, b_ref, o_ref):
        f32 = jnp.float32
        h = (sp_ref[0, :N].astype(f32) + sp_ref[1, :N].astype(f32)
             + t_ref[...].astype(f32)) * dis_ref[...]
        u = jnp.maximum(h * (g_ref[...] * BNC) + be_ref[...], 0.0)
        o = jnp.dot(u, w_ref[...], preferred_element_type=jnp.float32) + b_ref[...]
        if scale_out:
            o = (o * dis_ref[...]).astype(jnp.bfloat16)
        else:
            o = jnp.maximum(o, 0.0)
        o_ref[...] = o

    return pl.pallas_call(
        body,
        out_shape=jax.ShapeDtypeStruct(
            (N, d_out), jnp.bfloat16 if scale_out else jnp.float32),
    )(sp, t, dis, g, be, w, b)


def kernel(x, edge_index, W1, b1, g1, be1, W2, b2, g2, be2, W3, b3):
    row_p = edge_index[0].reshape(NW, NCH, C)
    col_p = edge_index[1].reshape(NW, NCH, C)

    degp = _sc_degree(col_p)
    t1, dis = _tc1(degp, x, W1, b1.reshape(1, 32))
    s1 = _sc_scatter32(t1, row_p, col_p)
    t2 = _tc_mid(s1, t1, dis, g1.reshape(1, 32),
                 be1.reshape(1, 32), W2, b2.reshape(1, 64), 64, True)
    s2 = _sc_scatter64(t2, row_p, col_p)
    out = _tc_mid(s2, t2, dis, g2.reshape(1, 64),
                  be2.reshape(1, 64), W3, b3.reshape(1, 128), 128, False)
    return out
